# baseline (speedup 1.0000x reference)
import functools

import jax
import jax.numpy as jnp
from jax import lax
from jax.experimental import pallas as pl
from jax.experimental.pallas import tpu as pltpu
from jax.experimental.pallas import tpu_sc as plsc

Q = 1024
K = 65536
D = 256
TOPK = 32
CHUNK = 128
NCHUNK = K // CHUNK
TK = 2048
NT = K // TK
CPT = TK // CHUNK
NCAND = TOPK * CHUNK
TQ4 = 256
BIG_I = 0x7FFFFFFF
INF_F = float("inf")


def _dist_block(q_ref, k_ref, qsq_ref, ksq_ref, dist_ref, cid_ref, m_acc):
    j = pl.program_id(0)
    q = q_ref[...]
    kt = k_ref[...]
    cross = lax.dot_general(
        q, kt, (((1,), (1,)), ((), ())),
        preferred_element_type=jnp.float32,
    )
    d2 = jnp.maximum(qsq_ref[...] + ksq_ref[...] - 2.0 * cross, 0.0)
    d3 = d2.reshape(Q, CPT, CHUNK)
    dist_ref[...] = d3
    mj = jnp.min(d3, axis=-1)

    for t in range(NT):
        @pl.when(j == t)
        def _(t=t):
            m_acc[:, t * CPT:(t + 1) * CPT] = mj

    @pl.when(j == NT - 1)
    def _():
        x = m_acc[...]
        idxl = lax.broadcasted_iota(jnp.int32, (Q, NCHUNK), 1)
        picks = []
        for _ in range(TOPK):
            rowmin = jnp.min(x, axis=1, keepdims=True)
            cand = jnp.where(x == rowmin, idxl, BIG_I)
            amin = jnp.min(cand, axis=1, keepdims=True)
            picks.append(amin)
            sel = idxl == amin
            x = jnp.where(sel, INF_F, x)
            idxl = jnp.where(sel, BIG_I, idxl)
        cid_ref[...] = jnp.concatenate(picks, axis=1)


def _distances_select(queries, keys):
    q_sq = jnp.sum(queries * queries, axis=-1, keepdims=True)
    k_sq = jnp.sum(keys * keys, axis=-1)[None, :]
    return pl.pallas_call(
        _dist_block,
        grid=(NT,),
        in_specs=[
            pl.BlockSpec((Q, D), lambda j: (0, 0)),
            pl.BlockSpec((TK, D), lambda j: (j, 0)),
            pl.BlockSpec((Q, 1), lambda j: (0, 0)),
            pl.BlockSpec((1, TK), lambda j: (0, j)),
        ],
        out_specs=[
            pl.BlockSpec((Q, CPT, CHUNK), lambda j: (0, j, 0)),
            pl.BlockSpec((Q, TOPK), lambda j: (0, 0)),
        ],
        out_shape=[
            jax.ShapeDtypeStruct((Q, NCHUNK, CHUNK), jnp.float32),
            jax.ShapeDtypeStruct((Q, TOPK), jnp.int32),
        ],
        scratch_shapes=[pltpu.VMEM((Q, NCHUNK), jnp.float32)],
    )(queries, keys, q_sq, k_sq)


def _make_sc_gather():
    info = plsc.get_sparse_core_info()
    nw = info.num_cores * info.num_subcores
    b = Q * TOPK
    b_per_w = b // nw
    sub = 512
    mesh = plsc.VectorSubcoreMesh(
        core_axis_name="c", subcore_axis_name="s")

    @functools.partial(
        pl.kernel, mesh=mesh,
        out_type=jax.ShapeDtypeStruct((b, CHUNK), jnp.float32),
        scratch_types=[
            pltpu.VMEM((sub,), jnp.int32),
            pltpu.VMEM((sub, CHUNK), jnp.float32),
            pltpu.SemaphoreType.DMA,
        ],
    )
    def gather_k(table_hbm, idx_hbm, out_hbm, idx_v, rows_v, sem):
        wid = lax.axis_index("s") * info.num_cores + lax.axis_index("c")
        base = wid * b_per_w
        for s in range(b_per_w // sub):
            off = base + s * sub
            pltpu.sync_copy(idx_hbm.at[pl.ds(off, sub)], idx_v)
            pltpu.async_copy(table_hbm.at[idx_v], rows_v, sem).wait()
            pltpu.sync_copy(rows_v, out_hbm.at[pl.ds(off, sub)])

    return gather_k


def _gather_chunks(dist3, chunk_idx):
    table = dist3.reshape(Q * NCHUNK, CHUNK)
    row_ids = (jnp.arange(Q, dtype=jnp.int32)[:, None] * NCHUNK
               + chunk_idx).reshape(-1)
    out = _make_sc_gather()(table, row_ids)
    return out.reshape(Q, NCAND)


def _final_body(v_ref, cid_ref, vals_ref, idx_ref):
    v = jnp.sqrt(v_ref[...] + 1e-12)
    cid = cid_ref[...]
    iota_c = lax.broadcasted_iota(jnp.int32, (TQ4, CHUNK), 1)
    gidx = jnp.concatenate(
        [cid[:, i:i + 1] * CHUNK + iota_c for i in range(TOPK)], axis=1)
    out_v, out_i = [], []
    for _ in range(TOPK):
        rowmin = jnp.min(v, axis=1, keepdims=True)
        cand = jnp.where(v == rowmin, gidx, BIG_I)
        amin = jnp.min(cand, axis=1, keepdims=True)
        out_v.append(rowmin)
        out_i.append(amin)
        sel = gidx == amin
        v = jnp.where(sel, INF_F, v)
        gidx = jnp.where(sel, BIG_I, gidx)
    vals_ref[...] = jnp.concatenate(out_v, axis=1)
    idx_ref[...] = jnp.concatenate(out_i, axis=1)


def _final_select(cands, chunk_idx):
    return pl.pallas_call(
        _final_body,
        grid=(Q // TQ4,),
        in_specs=[
            pl.BlockSpec((TQ4, NCAND), lambda i: (i, 0)),
            pl.BlockSpec((TQ4, TOPK), lambda i: (i, 0)),
        ],
        out_specs=[
            pl.BlockSpec((TQ4, TOPK), lambda i: (i, 0)),
            pl.BlockSpec((TQ4, TOPK), lambda i: (i, 0)),
        ],
        out_shape=(jax.ShapeDtypeStruct((Q, TOPK), jnp.float32),
                   jax.ShapeDtypeStruct((Q, TOPK), jnp.int32)),
    )(cands, chunk_idx)


def kernel(queries, keys, k):
    dist3, chunk_idx = _distances_select(queries, keys)
    cands = _gather_chunks(dist3, chunk_idx)
    vals, idx = _final_select(cands, chunk_idx)
    return (vals, idx)


# speedup vs baseline: 1.1540x; 1.1540x over previous
import functools

import jax
import jax.numpy as jnp
from jax import lax
from jax.experimental import pallas as pl
from jax.experimental.pallas import tpu as pltpu
from jax.experimental.pallas import tpu_sc as plsc

Q = 1024
K = 65536
D = 256
TOPK = 32
CHUNK = 128
NCHUNK = K // CHUNK
TK = 2048
CPT = TK // CHUNK
NCAND = TOPK * CHUNK
TQ4 = 256
BIG_I = 0x7FFFFFFF
INF_F = float("inf")


def _dist_block(q_ref, k_ref, qsq_ref, ksq_ref, dist_ref, m_ref):
    q = q_ref[...]
    kt = k_ref[...]
    cross = lax.dot_general(
        q, kt, (((1,), (1,)), ((), ())),
        preferred_element_type=jnp.float32,
    )
    d2 = jnp.maximum(qsq_ref[...] + ksq_ref[...] - 2.0 * cross, 0.0)
    d3 = d2.reshape(Q, CPT, CHUNK)
    dist_ref[...] = d3
    m_ref[0, ...] = jnp.min(d3, axis=-1)


def _distances(queries, keys):
    q_sq = jnp.sum(queries * queries, axis=-1, keepdims=True)
    k_sq = jnp.sum(keys * keys, axis=-1)[None, :]
    return pl.pallas_call(
        _dist_block,
        grid=(K // TK,),
        in_specs=[
            pl.BlockSpec((Q, D), lambda j: (0, 0)),
            pl.BlockSpec((TK, D), lambda j: (j, 0)),
            pl.BlockSpec((Q, 1), lambda j: (0, 0)),
            pl.BlockSpec((1, TK), lambda j: (0, j)),
        ],
        out_specs=[
            pl.BlockSpec((Q, CPT, CHUNK), lambda j: (0, j, 0)),
            pl.BlockSpec((1, Q, CPT), lambda j: (j, 0, 0)),
        ],
        out_shape=[
            jax.ShapeDtypeStruct((Q, NCHUNK, CHUNK), jnp.float32),
            jax.ShapeDtypeStruct((K // TK, Q, CPT), jnp.float32),
        ],
    )(queries, keys, q_sq, k_sq)


def _chunk_select_body(m_ref, out_ref):
    x = m_ref[...]
    idxl = lax.broadcasted_iota(jnp.int32, (Q, NCHUNK), 1)
    picks = []
    for _ in range(TOPK):
        rowmin = jnp.min(x, axis=1, keepdims=True)
        cand = jnp.where(x == rowmin, idxl, BIG_I)
        amin = jnp.min(cand, axis=1, keepdims=True)
        picks.append(amin)
        sel = idxl == amin
        x = jnp.where(sel, INF_F, x)
        idxl = jnp.where(sel, BIG_I, idxl)
    out_ref[...] = jnp.concatenate(picks, axis=1)


def _chunk_select(m):
    return pl.pallas_call(
        _chunk_select_body,
        out_shape=jax.ShapeDtypeStruct((Q, TOPK), jnp.int32),
    )(m)


def _make_sc_gather():
    info = plsc.get_sparse_core_info()
    nw = info.num_cores * info.num_subcores
    b = Q * TOPK
    b_per_w = b // nw
    sub = 512
    mesh = plsc.VectorSubcoreMesh(
        core_axis_name="c", subcore_axis_name="s")

    @functools.partial(
        pl.kernel, mesh=mesh,
        out_type=jax.ShapeDtypeStruct((b, CHUNK), jnp.float32),
        scratch_types=[
            pltpu.VMEM((sub,), jnp.int32),
            pltpu.VMEM((sub, CHUNK), jnp.float32),
            pltpu.SemaphoreType.DMA,
        ],
    )
    def gather_k(table_hbm, idx_hbm, out_hbm, idx_v, rows_v, sem):
        wid = lax.axis_index("s") * info.num_cores + lax.axis_index("c")
        base = wid * b_per_w
        for s in range(b_per_w // sub):
            off = base + s * sub
            pltpu.sync_copy(idx_hbm.at[pl.ds(off, sub)], idx_v)
            pltpu.async_copy(table_hbm.at[idx_v], rows_v, sem).wait()
            pltpu.sync_copy(rows_v, out_hbm.at[pl.ds(off, sub)])

    return gather_k


def _gather_chunks(dist3, chunk_idx):
    table = dist3.reshape(Q * NCHUNK, CHUNK)
    row_ids = (jnp.arange(Q, dtype=jnp.int32)[:, None] * NCHUNK
               + chunk_idx).reshape(-1)
    out = _make_sc_gather()(table, row_ids)
    return out.reshape(Q, NCAND)


def _final_body(v_ref, cid_ref, vals_ref, idx_ref):
    v = jnp.sqrt(v_ref[...] + 1e-12)
    cid = cid_ref[...]
    iota_c = lax.broadcasted_iota(jnp.int32, (TQ4, CHUNK), 1)
    gidx = jnp.concatenate(
        [cid[:, i:i + 1] * CHUNK + iota_c for i in range(TOPK)], axis=1)
    out_v, out_i = [], []
    for _ in range(TOPK):
        rowmin = jnp.min(v, axis=1, keepdims=True)
        cand = jnp.where(v == rowmin, gidx, BIG_I)
        amin = jnp.min(cand, axis=1, keepdims=True)
        out_v.append(rowmin)
        out_i.append(amin)
        sel = gidx == amin
        v = jnp.where(sel, INF_F, v)
        gidx = jnp.where(sel, BIG_I, gidx)
    vals_ref[...] = jnp.concatenate(out_v, axis=1)
    idx_ref[...] = jnp.concatenate(out_i, axis=1)


def _final_select(cands, chunk_idx):
    return pl.pallas_call(
        _final_body,
        grid=(Q // TQ4,),
        in_specs=[
            pl.BlockSpec((TQ4, NCAND), lambda i: (i, 0)),
            pl.BlockSpec((TQ4, TOPK), lambda i: (i, 0)),
        ],
        out_specs=[
            pl.BlockSpec((TQ4, TOPK), lambda i: (i, 0)),
            pl.BlockSpec((TQ4, TOPK), lambda i: (i, 0)),
        ],
        out_shape=(jax.ShapeDtypeStruct((Q, TOPK), jnp.float32),
                   jax.ShapeDtypeStruct((Q, TOPK), jnp.int32)),
    )(cands, chunk_idx)


def kernel(queries, keys, k):
    dist3, m3 = _distances(queries, keys)
    m = m3.transpose(1, 0, 2).reshape(Q, NCHUNK)
    chunk_idx = _chunk_select(m)
    cands = _gather_chunks(dist3, chunk_idx)
    vals, idx = _final_select(cands, chunk_idx)
    return (vals, idx)
